# baseline (speedup 1.0000x reference)
import functools

import jax
import jax.numpy as jnp
from jax import lax
from jax.experimental import pallas as pl
from jax.experimental.pallas import tpu as pltpu
from jax.experimental.pallas import tpu_sc as plsc

N_IN = 2048
N_TOTAL = 8192
C = 128
K = 4


_QB = 512


def _topk_body(q_ref, sh_ref, kx_ref, ky_ref, idx_ref):
    qx = q_ref[:, 0:1] - sh_ref[0:1, 0:1]
    qy = q_ref[:, 1:2] - sh_ref[0:1, 1:2]
    dx = qx - kx_ref[...]
    dy = qy - ky_ref[...]
    d = jnp.sqrt(dx * dx + dy * dy)
    iota_f = lax.broadcasted_iota(jnp.int32, (_QB, N_IN), 1).astype(jnp.float32)
    cols = []
    for _ in range(K):
        m = jnp.min(d, axis=1, keepdims=True)
        j = jnp.min(jnp.where(d == m, iota_f, jnp.float32(N_IN)),
                    axis=1, keepdims=True)
        d = jnp.where(iota_f == j, jnp.float32(jnp.inf), d)
        cols.append(j)
    idx_ref[...] = jnp.concatenate(cols, axis=1).astype(jnp.int32)


def _tc_topk(all_coords, shift2d, kx, ky):
    grid = all_coords.shape[0] // _QB
    return pl.pallas_call(
        _topk_body,
        grid=(grid,),
        in_specs=[
            pl.BlockSpec((_QB, 2), lambda i: (i, 0)),
            pl.BlockSpec((1, 2), lambda i: (0, 0)),
            pl.BlockSpec((1, N_IN), lambda i: (0, 0)),
            pl.BlockSpec((1, N_IN), lambda i: (0, 0)),
        ],
        out_specs=pl.BlockSpec((_QB, K), lambda i: (i, 0)),
        out_shape=jax.ShapeDtypeStruct((all_coords.shape[0], K), jnp.int32),
    )(all_coords, shift2d, kx, ky)


_NC = 2
_NS = 16
_NW = _NC * _NS
_QPW = N_TOTAL // (2 * _NW)
_QCHUNK = 32
_NCHUNK = _QPW // _QCHUNK


_GDEPTH = 4


def _gather_mean_body(table_hbm, idx_hbm, out_hbm,
                      idx_all, rows_v0, rows_v1, rows_v2, rows_v3,
                      out_v0, out_v1,
                      gsem0, gsem1, gsem2, gsem3, wsem0, wsem1):
    c = lax.axis_index("c")
    s = lax.axis_index("s")
    wid = s * _NC + c
    base_q = wid * _QPW
    row_bufs = (rows_v0, rows_v1, rows_v2, rows_v3)
    out_bufs = (out_v0, out_v1)
    gsems = (gsem0, gsem1, gsem2, gsem3)
    wsems = (wsem0, wsem1)

    pltpu.sync_copy(idx_hbm.at[pl.ds(base_q * K, _QPW * K)], idx_all)

    def start_gather(ch):
        p = ch % _GDEPTH
        isl = idx_all.at[pl.ds(ch * _QCHUNK * K, _QCHUNK * K)]
        return pltpu.async_copy(table_hbm.at[isl], row_bufs[p], gsems[p])

    gcop = [start_gather(ch) for ch in range(_GDEPTH)]
    wcop = [None, None]
    for ch in range(_NCHUNK):
        gcop[ch].wait()
        rows_v = row_bufs[ch % _GDEPTH]
        out_v = out_bufs[ch % 2]
        if wcop[ch % 2] is not None:
            wcop[ch % 2].wait()

        def q_body(q, carry2):
            for l in range(C // 16):
                sl = pl.ds(l * 16, 16)
                acc = (rows_v[K * q, sl] + rows_v[K * q + 1, sl]
                       + rows_v[K * q + 2, sl] + rows_v[K * q + 3, sl])
                out_v[q, sl] = acc * jnp.float32(1.0 / K)
            return carry2

        lax.fori_loop(0, _QCHUNK, q_body, 0, unroll=8)
        if ch + _GDEPTH < _NCHUNK:
            gcop.append(start_gather(ch + _GDEPTH))
        qb = base_q + ch * _QCHUNK
        wcop[ch % 2] = pltpu.async_copy(
            out_v, out_hbm.at[pl.ds(qb, _QCHUNK)], wsems[ch % 2])
    wcop[0].wait()
    wcop[1].wait()


def _sc_gather_mean(table, idx_flat):
    n_q = idx_flat.shape[0] // K
    mesh = plsc.VectorSubcoreMesh(core_axis_name="c", subcore_axis_name="s")
    kern = pl.kernel(
        _gather_mean_body,
        out_type=jax.ShapeDtypeStruct((n_q, C), jnp.float32),
        mesh=mesh,
        scratch_types=[
            pltpu.VMEM((_QPW * K,), jnp.int32),
            pltpu.VMEM((_QCHUNK * K, C), jnp.float32),
            pltpu.VMEM((_QCHUNK * K, C), jnp.float32),
            pltpu.VMEM((_QCHUNK * K, C), jnp.float32),
            pltpu.VMEM((_QCHUNK * K, C), jnp.float32),
            pltpu.VMEM((_QCHUNK, C), jnp.float32),
            pltpu.VMEM((_QCHUNK, C), jnp.float32),
            pltpu.SemaphoreType.DMA,
            pltpu.SemaphoreType.DMA,
            pltpu.SemaphoreType.DMA,
            pltpu.SemaphoreType.DMA,
            pltpu.SemaphoreType.DMA,
            pltpu.SemaphoreType.DMA,
        ],
    )
    return kern(table, idx_flat)


def kernel(values, coords, new_coords, shift):
    all_coords = jnp.concatenate([coords, new_coords], axis=0)
    kx = coords[:, 0][None, :]
    ky = coords[:, 1][None, :]
    table = values.T
    half = N_TOTAL // 2
    sh2 = shift[None, :]
    idx0 = _tc_topk(all_coords[:half], sh2, kx, ky)
    idx1 = _tc_topk(all_coords[half:], sh2, kx, ky)
    out0 = _sc_gather_mean(table, idx0.reshape(-1))
    out1 = _sc_gather_mean(table, idx1.reshape(-1))
    return jnp.concatenate([out0, out1], axis=0).T


# speedup vs baseline: 1.0218x; 1.0218x over previous
import functools

import jax
import jax.numpy as jnp
from jax import lax
from jax.experimental import pallas as pl
from jax.experimental.pallas import tpu as pltpu
from jax.experimental.pallas import tpu_sc as plsc

N_IN = 2048
N_TOTAL = 8192
C = 128
K = 4


_QB = 512


def _topk_body(q_ref, sh_ref, kx_ref, ky_ref, idx_ref):
    qx = q_ref[:, 0:1] - sh_ref[0:1, 0:1]
    qy = q_ref[:, 1:2] - sh_ref[0:1, 1:2]
    dx = qx - kx_ref[...]
    dy = qy - ky_ref[...]
    d = jnp.sqrt(dx * dx + dy * dy)
    iota_f = lax.broadcasted_iota(jnp.int32, (_QB, N_IN), 1).astype(jnp.float32)
    cols = []
    for _ in range(K):
        m = jnp.min(d, axis=1, keepdims=True)
        j = jnp.min(jnp.where(d == m, iota_f, jnp.float32(N_IN)),
                    axis=1, keepdims=True)
        d = jnp.where(iota_f == j, jnp.float32(jnp.inf), d)
        cols.append(j)
    idx_ref[...] = jnp.concatenate(cols, axis=1).astype(jnp.int32)


def _tc_topk(all_coords, shift2d, kx, ky):
    grid = all_coords.shape[0] // _QB
    return pl.pallas_call(
        _topk_body,
        grid=(grid,),
        in_specs=[
            pl.BlockSpec((_QB, 2), lambda i: (i, 0)),
            pl.BlockSpec((1, 2), lambda i: (0, 0)),
            pl.BlockSpec((1, N_IN), lambda i: (0, 0)),
            pl.BlockSpec((1, N_IN), lambda i: (0, 0)),
        ],
        out_specs=pl.BlockSpec((_QB, K), lambda i: (i, 0)),
        out_shape=jax.ShapeDtypeStruct((all_coords.shape[0], K), jnp.int32),
    )(all_coords, shift2d, kx, ky)


_NC = 2
_NS = 16
_NW = _NC * _NS
_QPW = N_TOTAL // (2 * _NW)
_QCHUNK = 32
_NCHUNK = _QPW // _QCHUNK


_GDEPTH = 3


def _gather_mean_body(table_hbm, idx_hbm, out_hbm,
                      idx_all, rows_v0, rows_v1, rows_v2,
                      out_v0, out_v1,
                      gsem0, gsem1, gsem2, wsem0, wsem1):
    c = lax.axis_index("c")
    s = lax.axis_index("s")
    wid = s * _NC + c
    base_q = wid * _QPW
    row_bufs = (rows_v0, rows_v1, rows_v2)
    out_bufs = (out_v0, out_v1)
    gsems = (gsem0, gsem1, gsem2)
    wsems = (wsem0, wsem1)

    pltpu.sync_copy(idx_hbm.at[pl.ds(base_q * K, _QPW * K)], idx_all)

    def start_gather(ch):
        p = ch % _GDEPTH
        isl = idx_all.at[pl.ds(ch * _QCHUNK * K, _QCHUNK * K)]
        return pltpu.async_copy(table_hbm.at[isl], row_bufs[p], gsems[p])

    gcop = [start_gather(ch) for ch in range(_GDEPTH)]
    wcop = [None, None]
    for ch in range(_NCHUNK):
        gcop[ch].wait()
        rows_v = row_bufs[ch % _GDEPTH]
        out_v = out_bufs[ch % 2]
        if wcop[ch % 2] is not None:
            wcop[ch % 2].wait()

        def q_body(q, carry2):
            for l in range(C // 16):
                sl = pl.ds(l * 16, 16)
                acc = (rows_v[K * q, sl] + rows_v[K * q + 1, sl]
                       + rows_v[K * q + 2, sl] + rows_v[K * q + 3, sl])
                out_v[q, sl] = acc * jnp.float32(1.0 / K)
            return carry2

        lax.fori_loop(0, _QCHUNK, q_body, 0, unroll=4)
        if ch + _GDEPTH < _NCHUNK:
            gcop.append(start_gather(ch + _GDEPTH))
        qb = base_q + ch * _QCHUNK
        wcop[ch % 2] = pltpu.async_copy(
            out_v, out_hbm.at[pl.ds(qb, _QCHUNK)], wsems[ch % 2])
    wcop[0].wait()
    wcop[1].wait()


def _sc_gather_mean(table, idx_flat):
    n_q = idx_flat.shape[0] // K
    mesh = plsc.VectorSubcoreMesh(core_axis_name="c", subcore_axis_name="s")
    kern = pl.kernel(
        _gather_mean_body,
        out_type=jax.ShapeDtypeStruct((n_q, C), jnp.float32),
        mesh=mesh,
        scratch_types=[
            pltpu.VMEM((_QPW * K,), jnp.int32),
            pltpu.VMEM((_QCHUNK * K, C), jnp.float32),
            pltpu.VMEM((_QCHUNK * K, C), jnp.float32),
            pltpu.VMEM((_QCHUNK * K, C), jnp.float32),
            pltpu.VMEM((_QCHUNK, C), jnp.float32),
            pltpu.VMEM((_QCHUNK, C), jnp.float32),
            pltpu.SemaphoreType.DMA,
            pltpu.SemaphoreType.DMA,
            pltpu.SemaphoreType.DMA,
            pltpu.SemaphoreType.DMA,
            pltpu.SemaphoreType.DMA,
        ],
    )
    return kern(table, idx_flat)


def kernel(values, coords, new_coords, shift):
    all_coords = jnp.concatenate([coords, new_coords], axis=0)
    kx = coords[:, 0][None, :]
    ky = coords[:, 1][None, :]
    table = values.T
    half = N_TOTAL // 2
    sh2 = shift[None, :]
    idx0 = _tc_topk(all_coords[:half], sh2, kx, ky)
    idx1 = _tc_topk(all_coords[half:], sh2, kx, ky)
    out0 = _sc_gather_mean(table, idx0.reshape(-1))
    out1 = _sc_gather_mean(table, idx1.reshape(-1))
    return jnp.concatenate([out0, out1], axis=0).T


# speedup vs baseline: 1.0227x; 1.0008x over previous
import jax
import jax.numpy as jnp
from jax import lax
from jax.experimental import pallas as pl
from jax.experimental.pallas import tpu as pltpu
from jax.experimental.pallas import tpu_sc as plsc

N_IN = 2048
N_TOTAL = 8192
C = 128
K = 4


_QB = 512


def _topk_body(q_ref, sh_ref, kx_ref, ky_ref, idx_ref):
    qx = q_ref[:, 0:1] - sh_ref[0:1, 0:1]
    qy = q_ref[:, 1:2] - sh_ref[0:1, 1:2]
    dx = qx - kx_ref[...]
    dy = qy - ky_ref[...]
    d = jnp.sqrt(dx * dx + dy * dy)
    iota_f = lax.broadcasted_iota(jnp.int32, (_QB, N_IN), 1).astype(jnp.float32)
    cols = []
    for _ in range(K):
        m = jnp.min(d, axis=1, keepdims=True)
        j = jnp.min(jnp.where(d == m, iota_f, jnp.float32(N_IN)),
                    axis=1, keepdims=True)
        d = jnp.where(iota_f == j, jnp.float32(jnp.inf), d)
        cols.append(j)
    idx_ref[...] = jnp.concatenate(cols, axis=1).astype(jnp.int32)


def _tc_topk(all_coords, shift2d, kx, ky):
    grid = all_coords.shape[0] // _QB
    return pl.pallas_call(
        _topk_body,
        grid=(grid,),
        in_specs=[
            pl.BlockSpec((_QB, 2), lambda i: (i, 0)),
            pl.BlockSpec((1, 2), lambda i: (0, 0)),
            pl.BlockSpec((1, N_IN), lambda i: (0, 0)),
            pl.BlockSpec((1, N_IN), lambda i: (0, 0)),
        ],
        out_specs=pl.BlockSpec((_QB, K), lambda i: (i, 0)),
        out_shape=jax.ShapeDtypeStruct((all_coords.shape[0], K), jnp.int32),
    )(all_coords, shift2d, kx, ky)


_NC = 2
_NS = 16
_NW = _NC * _NS
_QPW = N_TOTAL // (2 * _NW)
_QCHUNK = 32
_NCHUNK = _QPW // _QCHUNK


_GDEPTH = 3


def _gather_mean_body(table_hbm, idx_hbm, out_hbm,
                      idx_all, rows_v0, rows_v1, rows_v2,
                      out_v0, out_v1,
                      gsem0, gsem1, gsem2, wsem0, wsem1):
    c = lax.axis_index("c")
    s = lax.axis_index("s")
    wid = s * _NC + c
    base_q = wid * _QPW
    row_bufs = (rows_v0, rows_v1, rows_v2)
    out_bufs = (out_v0, out_v1)
    gsems = (gsem0, gsem1, gsem2)
    wsems = (wsem0, wsem1)

    pltpu.sync_copy(idx_hbm.at[pl.ds(base_q * K, _QPW * K)], idx_all)

    def start_gather(ch):
        p = ch % _GDEPTH
        isl = idx_all.at[pl.ds(ch * _QCHUNK * K, _QCHUNK * K)]
        return pltpu.async_copy(table_hbm.at[isl], row_bufs[p], gsems[p])

    gcop = [start_gather(ch) for ch in range(_GDEPTH)]
    wcop = [None, None]
    for ch in range(_NCHUNK):
        gcop[ch].wait()
        rows_v = row_bufs[ch % _GDEPTH]
        out_v = out_bufs[ch % 2]
        if wcop[ch % 2] is not None:
            wcop[ch % 2].wait()

        def q_body(q, carry2):
            for l in range(C // 16):
                sl = pl.ds(l * 16, 16)
                acc = (rows_v[K * q, sl] + rows_v[K * q + 1, sl]
                       + rows_v[K * q + 2, sl] + rows_v[K * q + 3, sl])
                out_v[q, sl] = acc * jnp.float32(1.0 / K)
            return carry2

        lax.fori_loop(0, _QCHUNK, q_body, 0, unroll=4)
        if ch + _GDEPTH < _NCHUNK:
            gcop.append(start_gather(ch + _GDEPTH))
        qb = base_q + ch * _QCHUNK
        wcop[ch % 2] = pltpu.async_copy(
            out_v, out_hbm.at[pl.ds(qb, _QCHUNK)], wsems[ch % 2])
    wcop[0].wait()
    wcop[1].wait()


def _sc_gather_mean(table, idx_flat):
    n_q = idx_flat.shape[0] // K
    mesh = plsc.VectorSubcoreMesh(core_axis_name="c", subcore_axis_name="s")
    kern = pl.kernel(
        _gather_mean_body,
        out_type=jax.ShapeDtypeStruct((n_q, C), jnp.float32),
        mesh=mesh,
        scratch_types=[
            pltpu.VMEM((_QPW * K,), jnp.int32),
            pltpu.VMEM((_QCHUNK * K, C), jnp.float32),
            pltpu.VMEM((_QCHUNK * K, C), jnp.float32),
            pltpu.VMEM((_QCHUNK * K, C), jnp.float32),
            pltpu.VMEM((_QCHUNK, C), jnp.float32),
            pltpu.VMEM((_QCHUNK, C), jnp.float32),
            pltpu.SemaphoreType.DMA,
            pltpu.SemaphoreType.DMA,
            pltpu.SemaphoreType.DMA,
            pltpu.SemaphoreType.DMA,
            pltpu.SemaphoreType.DMA,
        ],
    )
    return kern(table, idx_flat)


def kernel(values, coords, new_coords, shift):
    all_coords = jnp.concatenate([coords, new_coords], axis=0)
    kx = coords[:, 0][None, :]
    ky = coords[:, 1][None, :]
    table = values.T
    half = N_TOTAL // 2
    sh2 = shift[None, :]
    idx0 = _tc_topk(all_coords[:half], sh2, kx, ky)
    idx1 = _tc_topk(all_coords[half:], sh2, kx, ky)
    out0 = _sc_gather_mean(table, idx0.reshape(-1))
    out1 = _sc_gather_mean(table, idx1.reshape(-1))
    return jnp.concatenate([out0, out1], axis=0).T
